# baseline (speedup 1.0000x reference)
import functools

import jax
import jax.numpy as jnp
from jax import lax
from jax.experimental import pallas as pl
from jax.experimental.pallas import tpu as pltpu
from jax.experimental.pallas import tpu_sc as plsc

_N = 10000
_F = 128
_NP = 10240
_NC = 2
_NS = 16
_NW = _NC * _NS
_LN = 16
_NPT = _NP // _NS
_BLK = 8


def _mesh():
    return plsc.VectorSubcoreMesh(core_axis_name="c", subcore_axis_name="s")


_SC_PARAMS = pltpu.CompilerParams(needs_layout_passes=False)


def _msg_body(rpt, xw, src2d, dst2d, w2d, out, dis_out,
              srcA, dstA, srcB, dstB, wvb, disv, rowsA, rowsB, zbuf,
              degacc, acc, sem_st, sem_w, sem_gA, sem_gB, sem_sA, sem_sB):
    c = lax.axis_index("c")
    s = lax.axis_index("s")
    t = c * _NS + s
    nblk = rpt // _BLK
    tr = _NW * rpt
    trs = tr // _NS
    dblk = trs // _BLK

    for i in range(128 // _LN):
        zbuf[pl.ds(i * _LN, _LN)] = jnp.zeros((_LN,), jnp.float32)
    for k in range(_NPT // 128):
        pltpu.sync_copy(zbuf, degacc.at[pl.ds(s * _NPT + k * 128, 128)])
    wbase = pl.multiple_of(s * trs, _BLK)
    half = trs // 2
    pltpu.sync_copy(w2d.at[pl.ds(wbase, half)], rowsA.at[pl.ds(0, half)])
    pltpu.sync_copy(w2d.at[pl.ds(wbase + half, half)],
                    rowsB.at[pl.ds(0, half)])
    plsc.subcore_barrier()

    def dbase(b):
        return pl.multiple_of(s * trs + b * _BLK, _BLK)

    def deg_stage(b, ib):
        pltpu.async_copy(dst2d.at[pl.ds(dbase(b), _BLK)], ib, sem_st)

    def deg_wait(b, ib):
        pltpu.make_async_copy(dst2d.at[pl.ds(dbase(b), _BLK)], ib,
                              sem_st).wait()

    deg_stage(0, srcA)

    def deg_block(b, ib, nb, vrows, voff):
        deg_wait(b, ib)

        @pl.when(b + 1 < dblk)
        def _():
            deg_stage(b + 1, nb)

        for j in range(_BLK):
            vr = b * _BLK + j - voff
            pltpu.async_copy(vrows.at[vr], degacc.at[ib.at[j]], sem_w,
                             add=True)
        for j in range(_BLK):
            pltpu.make_async_copy(vrows.at[0], degacc.at[ib.at[0]],
                                  sem_w).wait()

    def deg_pair_a(hp, carry):
        b = hp * 2
        deg_block(b, srcA, srcB, rowsA, 0)
        deg_block(b + 1, srcB, srcA, rowsA, 0)
        return carry

    def deg_pair_b(hp, carry):
        b = (dblk // 2) + hp * 2
        deg_block(b, srcA, srcB, rowsB, half)
        deg_block(b + 1, srcB, srcA, rowsB, half)
        return carry

    lax.fori_loop(0, dblk // 4, deg_pair_a, 0)
    lax.fori_loop(0, dblk // 4, deg_pair_b, 0)
    plsc.subcore_barrier()

    pltpu.sync_copy(degacc, disv)

    def nw(i, carry):
        sl = pl.ds(i * _LN, _LN)
        d = disv[sl] + 1.0
        yi = jnp.int32(0x5F3759DF) - lax.shift_right_logical(
            plsc.bitcast(d, jnp.int32), 1)
        y = plsc.bitcast(yi, jnp.float32)
        hd = 0.5 * d
        for _ in range(3):
            y = y * (1.5 - hd * y * y)
        disv[sl] = y
        return carry

    lax.fori_loop(0, _NP // _LN, nw, 0)

    @pl.when(c == 0)
    def _():
        pltpu.sync_copy(disv.at[pl.ds(s * _NPT, _NPT)],
                        dis_out.at[pl.ds(s * _NPT, _NPT)])

    def zr(r, carry):
        for cc in range(8):
            rowsA[r, pl.ds(cc * _LN, _LN)] = jnp.zeros((_LN,), jnp.float32)
        return carry

    lax.fori_loop(0, 128, zr, 0)
    for b in range(_NPT // 128):
        pltpu.sync_copy(rowsA, acc.at[pl.ds(s * _NPT + b * 128, 128)])
    plsc.subcore_barrier()

    def ebase(bi):
        return pl.multiple_of(t * rpt + bi * _BLK, _BLK)

    def issue_stage(bi, sb, db):
        pltpu.async_copy(src2d.at[pl.ds(ebase(bi), _BLK)], sb, sem_st)
        pltpu.async_copy(dst2d.at[pl.ds(ebase(bi), _BLK)], db, sem_st)

    def wait_stage(bi, sb, db):
        pltpu.make_async_copy(src2d.at[pl.ds(ebase(bi), _BLK)], sb,
                              sem_st).wait()
        pltpu.make_async_copy(dst2d.at[pl.ds(ebase(bi), _BLK)], db,
                              sem_st).wait()

    def issue_wv(bi):
        pltpu.async_copy(w2d.at[pl.ds(ebase(bi), _BLK)], wvb, sem_w)

    def wait_wv(bi):
        pltpu.make_async_copy(w2d.at[pl.ds(ebase(bi), _BLK)], wvb,
                              sem_w).wait()

    issue_stage(0, srcA, dstA)
    issue_wv(0)
    wait_stage(0, srcA, dstA)
    pltpu.async_copy(xw.at[srcA.at[0]], rowsA, sem_gA)

    def wait_scat(which):
        rowsX, semX = (rowsA, sem_sA) if which == 0 else (rowsB, sem_sB)
        pltpu.make_async_copy(rowsX, acc.at[dstA.at[0]], semX).wait()

    def do_row(bi, rp, r, even, srcC, dstC, srcN):
        rowsC, semC = (rowsA, sem_gA) if even else (rowsB, sem_gB)
        rowsO, semO = (rowsB, sem_gB) if even else (rowsA, sem_gA)
        semSC = sem_sA if even else sem_sB
        pltpu.make_async_copy(xw.at[srcC.at[r]], rowsC, semC).wait()
        if even:
            @pl.when(bi + rp > 0)
            def _():
                wait_scat(1)

            pltpu.async_copy(xw.at[srcC.at[r + 1]], rowsO, semO)
        else:
            @pl.when(r < _BLK - 1)
            def _():
                wait_scat(0)
                pltpu.async_copy(xw.at[srcC.at[r + 1]], rowsO, semO)

            @pl.when(jnp.logical_and(r >= _BLK - 1, bi + 1 < nblk))
            def _():
                wait_scat(0)
                wait_stage(bi + 1, srcN[0], srcN[1])
                pltpu.async_copy(xw.at[srcN[0].at[0]], rowsO, semO)

        def sg(g, carry):
            sl = pl.ds(g * _LN, _LN)
            av = plsc.load_gather(disv, [srcC[r, sl]])
            bv = plsc.load_gather(disv, [dstC[r, sl]])
            nvv = av * wvb[r, sl] * bv
            for i in range(_LN):
                nv = nvv[i]
                rr = g * _LN + i
                for cc in range(8):
                    s2 = pl.ds(cc * _LN, _LN)
                    rowsC[rr, s2] = rowsC[rr, s2] * nv
            return carry

        lax.fori_loop(0, 8, sg, 0)
        pltpu.async_copy(rowsC, acc.at[dstC.at[r]], semSC, add=True)

    def do_block(bi, srcC, dstC, srcN, dstN):
        @pl.when(bi + 1 < nblk)
        def _():
            issue_stage(bi + 1, srcN, dstN)

        wait_wv(bi)

        def rp_body(rp, carry):
            do_row(bi, rp, rp * 2, True, srcC, dstC, (srcN, dstN))
            do_row(bi, rp, rp * 2 + 1, False, srcC, dstC, (srcN, dstN))
            return carry

        lax.fori_loop(0, _BLK // 2, rp_body, 0)

        @pl.when(bi + 1 < nblk)
        def _():
            issue_wv(bi + 1)

    def bp_body(bp, carry):
        do_block(bp * 2, srcA, dstA, srcB, dstB)
        do_block(bp * 2 + 1, srcB, dstB, srcA, dstA)
        return carry

    lax.fori_loop(0, nblk // 2, bp_body, 0)
    wait_scat(0)
    wait_scat(1)
    plsc.subcore_barrier()
    pltpu.sync_copy(acc.at[pl.ds(s * _NPT, _NPT)],
                    out.at[c, pl.ds(s * _NPT, _NPT)])


def _msg_call(xw, src2d, dst2d, w2d, rpt):
    kfn = pl.kernel(
        functools.partial(_msg_body, rpt),
        out_type=[
            jax.ShapeDtypeStruct((_NC, _NP, _F), jnp.float32),
            jax.ShapeDtypeStruct((_NP,), jnp.float32),
        ],
        mesh=_mesh(),
        compiler_params=_SC_PARAMS,
        scratch_types=[
            pltpu.VMEM((_BLK, 128), jnp.int32),
            pltpu.VMEM((_BLK, 128), jnp.int32),
            pltpu.VMEM((_BLK, 128), jnp.int32),
            pltpu.VMEM((_BLK, 128), jnp.int32),
            pltpu.VMEM((_BLK, 128), jnp.float32),
            pltpu.VMEM((_NP,), jnp.float32),
            pltpu.VMEM((128, _F), jnp.float32),
            pltpu.VMEM((128, _F), jnp.float32),
            pltpu.VMEM((128,), jnp.float32),
            pltpu.VMEM_SHARED((_NP,), jnp.float32),
            pltpu.VMEM_SHARED((_NP, _F), jnp.float32),
            pltpu.SemaphoreType.DMA,
            pltpu.SemaphoreType.DMA,
            pltpu.SemaphoreType.DMA,
            pltpu.SemaphoreType.DMA,
            pltpu.SemaphoreType.DMA,
            pltpu.SemaphoreType.DMA,
        ],
    )
    return kfn(xw, src2d, dst2d, w2d)


def _dense_body(xflat_ref, pblk_ref, p_ref, xp_ref, wih_ref, whh_ref,
                bih_ref, bhh_ref, w0_ref, xw_ref, xt_scr):
    pv = p_ref[...]
    inv = lax.rsqrt(jnp.sum(pv * pv))
    s = jnp.dot(xflat_ref[...], pblk_ref[...],
                preferred_element_type=jnp.float32) * inv

    nr = _NP // 128
    ri = lax.broadcasted_iota(jnp.int32, (nr, 128), 0)
    ci = lax.broadcasted_iota(jnp.int32, (nr, 128), 1)
    flat = ri * 128 + ci
    s = jnp.where(flat >= _N, -jnp.inf, s)

    def body(k, s):
        m = jnp.max(s)
        fi = jnp.min(jnp.where(s >= m, flat, jnp.int32(2**30)))
        row = xp_ref[pl.ds(fi, 1), :]
        gate = jnp.tanh(jnp.broadcast_to(m, (1, _F)))
        xt_scr[pl.ds(k, 1), :] = row * gate
        return jnp.where(flat == fi, -jnp.inf, s)

    lax.fori_loop(0, _F, body, s)

    xt = xt_scr[...]
    gi = jnp.dot(xt, wih_ref[...], preferred_element_type=jnp.float32)
    gi = gi + bih_ref[...]
    gh = jnp.dot(w0_ref[...], whh_ref[...], preferred_element_type=jnp.float32)
    gh = gh + bhh_ref[...]
    r = jax.nn.sigmoid(gi[:, :_F] + gh[:, :_F])
    z = jax.nn.sigmoid(gi[:, _F:2 * _F] + gh[:, _F:2 * _F])
    n = jnp.tanh(gi[:, 2 * _F:] + r * gh[:, 2 * _F:])
    w = (1.0 - z) * n + z * w0_ref[...]
    xw_ref[...] = jnp.dot(xp_ref[...], w, preferred_element_type=jnp.float32)


def _dense_call(xflat, pblk, p2d, xp, wihT, whhT, bih2, bhh2, w0):
    return pl.pallas_call(
        _dense_body,
        out_shape=jax.ShapeDtypeStruct((_NP, _F), jnp.float32),
        scratch_shapes=[pltpu.VMEM((_F, _F), jnp.float32)],
    )(xflat, pblk, p2d, xp, wihT, whhT, bih2, bhh2, w0)


def _final_body(acc_ref, xw_ref, dis_ref, wl_ref, bl_ref, o_ref):
    dv = dis_ref[...]
    out = acc_ref[0] + acc_ref[1] + (dv * dv) * xw_ref[...]
    h = jnp.maximum(out, 0.0)
    o_ref[...] = jnp.dot(h, wl_ref[...],
                         preferred_element_type=jnp.float32) + bl_ref[...]


def _final_call(acc, xw, dis2d, wlT, bl2):
    return pl.pallas_call(
        _final_body,
        out_shape=jax.ShapeDtypeStruct((_NP, wlT.shape[1]), jnp.float32),
    )(acc, xw, dis2d, wlT, bl2)


def kernel(x, edge_index, edge_weight, p, W_ih, W_hh, b_ih, b_hh, W0,
           W_lin, b_lin):
    n, f = x.shape
    e = edge_weight.shape[0]
    rpt = -(-e // (_NW * 128))
    rpt = -(-rpt // (2 * _BLK)) * (2 * _BLK)
    e_pad = _NW * 128 * rpt
    fill = jnp.arange(e_pad - e, dtype=jnp.int32) % n
    src = jnp.concatenate([edge_index[0], fill])
    dst = jnp.concatenate([edge_index[1], fill])
    w = jnp.concatenate([edge_weight,
                         jnp.zeros((e_pad - e,), edge_weight.dtype)])
    tr = _NW * rpt
    src2d = src.reshape(tr, 128)
    dst2d = dst.reshape(tr, 128)
    w2d = w.reshape(tr, 128)

    xp = jnp.pad(x, ((0, _NP - n), (0, 0)))
    xflat = xp.reshape(_NP // 128, 128 * f)
    pblk = (jnp.eye(f, dtype=jnp.float32)[:, None, :]
            * p[None, :, None]).reshape(f * f, f)

    xw = _dense_call(xflat, pblk, p.reshape(1, f), xp,
                     W_ih.T, W_hh.T, b_ih.reshape(1, -1),
                     b_hh.reshape(1, -1), W0)

    acc, dis = _msg_call(xw, src2d, dst2d, w2d, rpt)

    y = _final_call(acc, xw, dis.reshape(_NP, 1), W_lin.T,
                    b_lin.reshape(1, -1))
    return y[:n]


# speedup vs baseline: 1.1349x; 1.1349x over previous
import functools

import jax
import jax.numpy as jnp
from jax import lax
from jax.experimental import pallas as pl
from jax.experimental.pallas import tpu as pltpu
from jax.experimental.pallas import tpu_sc as plsc

_N = 10000
_F = 128
_NP = 10240
_NC = 2
_NS = 16
_NW = _NC * _NS
_LN = 16
_NPT = _NP // _NS
_BLK = 8


def _mesh():
    return plsc.VectorSubcoreMesh(core_axis_name="c", subcore_axis_name="s")


_SC_PARAMS = pltpu.CompilerParams(needs_layout_passes=False)


def _deg_body(rpt, dst2d, w2d, out, dstv, wv, degacc, zbuf):
    c = lax.axis_index("c")
    s = lax.axis_index("s")
    t = c * _NS + s
    for i in range(_NPT // _LN):
        zbuf[pl.ds(i * _LN, _LN)] = jnp.zeros((_LN,), jnp.float32)
    pltpu.sync_copy(zbuf, degacc.at[pl.ds(s * _NPT, _NPT)])
    pltpu.sync_copy(dst2d.at[pl.ds(t * rpt, rpt)], dstv)
    pltpu.sync_copy(w2d.at[pl.ds(t * rpt, rpt)], wv)
    plsc.subcore_barrier()

    def step(j, carry):
        pltpu.sync_copy(wv.at[j], degacc.at[dstv.at[j]], add=True)
        return carry

    lax.fori_loop(0, rpt, step, 0)
    plsc.subcore_barrier()
    pltpu.sync_copy(degacc.at[pl.ds(s * _NPT, _NPT)],
                    out.at[c, pl.ds(s * _NPT, _NPT)])


def _deg_call(dst2d, w2d, rpt):
    kfn = pl.kernel(
        functools.partial(_deg_body, rpt),
        out_type=jax.ShapeDtypeStruct((_NC, _NP), jnp.float32),
        mesh=_mesh(),
        compiler_params=_SC_PARAMS,
        scratch_types=[
            pltpu.VMEM((rpt, 128), jnp.int32),
            pltpu.VMEM((rpt, 128), jnp.float32),
            pltpu.VMEM_SHARED((_NP,), jnp.float32),
            pltpu.VMEM((_NPT,), jnp.float32),
        ],
    )
    return kfn(dst2d, w2d)


def _msg_body(rpt, xw, dis, src2d, dst2d, w2d, out,
              srcA, dstA, srcB, dstB, wvb, disv, rowsA, rowsB, acc,
              sem_st, sem_w, sem_gA, sem_gB, sem_sA, sem_sB):
    c = lax.axis_index("c")
    s = lax.axis_index("s")
    t = c * _NS + s
    nblk = rpt // _BLK

    def ebase(bi):
        return pl.multiple_of(t * rpt + bi * _BLK, _BLK)

    def issue_stage(bi, sb, db):
        pltpu.async_copy(src2d.at[pl.ds(ebase(bi), _BLK)], sb, sem_st)
        pltpu.async_copy(dst2d.at[pl.ds(ebase(bi), _BLK)], db, sem_st)

    def wait_stage(bi, sb, db):
        pltpu.make_async_copy(src2d.at[pl.ds(ebase(bi), _BLK)], sb,
                              sem_st).wait()
        pltpu.make_async_copy(dst2d.at[pl.ds(ebase(bi), _BLK)], db,
                              sem_st).wait()

    def issue_wv(bi):
        pltpu.async_copy(w2d.at[pl.ds(ebase(bi), _BLK)], wvb, sem_w)

    def wait_wv(bi):
        pltpu.make_async_copy(w2d.at[pl.ds(ebase(bi), _BLK)], wvb,
                              sem_w).wait()

    def zr(r, carry):
        for cc in range(8):
            rowsA[r, pl.ds(cc * _LN, _LN)] = jnp.zeros((_LN,), jnp.float32)
        return carry

    lax.fori_loop(0, 128, zr, 0)
    for b in range(_NPT // 128):
        pltpu.sync_copy(rowsA, acc.at[pl.ds(s * _NPT + b * 128, 128)])

    pltpu.sync_copy(dis, disv)
    plsc.subcore_barrier()

    issue_stage(0, srcA, dstA)
    issue_wv(0)
    wait_stage(0, srcA, dstA)
    pltpu.async_copy(xw.at[srcA.at[0]], rowsA, sem_gA)

    def wait_scat(which):
        rowsX, semX = (rowsA, sem_sA) if which == 0 else (rowsB, sem_sB)
        pltpu.make_async_copy(rowsX, acc.at[dstA.at[0]], semX).wait()

    def do_row(bi, rp, r, even, srcC, dstC, srcN):
        rowsC, semC = (rowsA, sem_gA) if even else (rowsB, sem_gB)
        rowsO, semO = (rowsB, sem_gB) if even else (rowsA, sem_gA)
        semSC = sem_sA if even else sem_sB
        pltpu.make_async_copy(xw.at[srcC.at[r]], rowsC, semC).wait()
        if even:
            @pl.when(bi + rp > 0)
            def _():
                wait_scat(1)

            pltpu.async_copy(xw.at[srcC.at[r + 1]], rowsO, semO)
        else:
            @pl.when(r < _BLK - 1)
            def _():
                wait_scat(0)
                pltpu.async_copy(xw.at[srcC.at[r + 1]], rowsO, semO)

            @pl.when(jnp.logical_and(r >= _BLK - 1, bi + 1 < nblk))
            def _():
                wait_scat(0)
                wait_stage(bi + 1, srcN[0], srcN[1])
                pltpu.async_copy(xw.at[srcN[0].at[0]], rowsO, semO)

        def sg(g, carry):
            sl = pl.ds(g * _LN, _LN)
            av = plsc.load_gather(disv, [srcC[r, sl]])
            bv = plsc.load_gather(disv, [dstC[r, sl]])
            nvv = av * wvb[r, sl] * bv
            for i in range(_LN):
                nv = nvv[i]
                rr = g * _LN + i
                for cc in range(8):
                    s2 = pl.ds(cc * _LN, _LN)
                    rowsC[rr, s2] = rowsC[rr, s2] * nv
            return carry

        lax.fori_loop(0, 8, sg, 0)
        pltpu.async_copy(rowsC, acc.at[dstC.at[r]], semSC, add=True)

    def do_block(bi, srcC, dstC, srcN, dstN):
        @pl.when(bi + 1 < nblk)
        def _():
            issue_stage(bi + 1, srcN, dstN)

        wait_wv(bi)

        def rp_body(rp, carry):
            do_row(bi, rp, rp * 2, True, srcC, dstC, (srcN, dstN))
            do_row(bi, rp, rp * 2 + 1, False, srcC, dstC, (srcN, dstN))
            return carry

        lax.fori_loop(0, _BLK // 2, rp_body, 0)

        @pl.when(bi + 1 < nblk)
        def _():
            issue_wv(bi + 1)

    def bp_body(bp, carry):
        do_block(bp * 2, srcA, dstA, srcB, dstB)
        do_block(bp * 2 + 1, srcB, dstB, srcA, dstA)
        return carry

    lax.fori_loop(0, nblk // 2, bp_body, 0)
    wait_scat(0)
    wait_scat(1)
    plsc.subcore_barrier()
    pltpu.sync_copy(acc.at[pl.ds(s * _NPT, _NPT)],
                    out.at[c, pl.ds(s * _NPT, _NPT)])


def _msg_call(xw, dis, src2d, dst2d, w2d, rpt):
    kfn = pl.kernel(
        functools.partial(_msg_body, rpt),
        out_type=jax.ShapeDtypeStruct((_NC, _NP, _F), jnp.float32),
        mesh=_mesh(),
        compiler_params=_SC_PARAMS,
        scratch_types=[
            pltpu.VMEM((_BLK, 128), jnp.int32),
            pltpu.VMEM((_BLK, 128), jnp.int32),
            pltpu.VMEM((_BLK, 128), jnp.int32),
            pltpu.VMEM((_BLK, 128), jnp.int32),
            pltpu.VMEM((_BLK, 128), jnp.float32),
            pltpu.VMEM((_NP,), jnp.float32),
            pltpu.VMEM((128, _F), jnp.float32),
            pltpu.VMEM((128, _F), jnp.float32),
            pltpu.VMEM_SHARED((_NP, _F), jnp.float32),
            pltpu.SemaphoreType.DMA,
            pltpu.SemaphoreType.DMA,
            pltpu.SemaphoreType.DMA,
            pltpu.SemaphoreType.DMA,
            pltpu.SemaphoreType.DMA,
            pltpu.SemaphoreType.DMA,
        ],
    )
    return kfn(xw, dis, src2d, dst2d, w2d)


def _score_body(x_ref, p_ref, o_ref):
    pv = p_ref[...]
    inv = lax.rsqrt(jnp.sum(pv * pv))
    o_ref[...] = jnp.dot(x_ref[...], pv,
                         preferred_element_type=jnp.float32) * inv


def _score_call(x, p2d):
    return pl.pallas_call(
        _score_body,
        out_shape=jax.ShapeDtypeStruct((x.shape[0], 1), jnp.float32),
    )(x, p2d)


def _dense_body(s_ref, xp_ref, dp_ref, wih_ref, whh_ref, bih_ref, bhh_ref,
                w0_ref, xw_ref, dis_ref, sw_ref, xt_scr):
    d = dp_ref[0] + dp_ref[1] + 1.0
    dis_ref[...] = lax.rsqrt(d)
    sw_ref[...] = 1.0 / d

    nr = _NP // 128
    ri = lax.broadcasted_iota(jnp.int32, (nr, 128), 0)
    ci = lax.broadcasted_iota(jnp.int32, (nr, 128), 1)
    flat = ri * 128 + ci
    s = jnp.where(flat >= _N, -jnp.inf, s_ref[...])

    def body(k, s):
        m = jnp.max(s)
        fi = jnp.min(jnp.where(s >= m, flat, jnp.int32(2**30)))
        row = xp_ref[pl.ds(fi, 1), :]
        gate = jnp.tanh(jnp.broadcast_to(m, (1, _F)))
        xt_scr[pl.ds(k, 1), :] = row * gate
        return jnp.where(flat == fi, -jnp.inf, s)

    lax.fori_loop(0, _F, body, s)

    xt = xt_scr[...]
    gi = jnp.dot(xt, wih_ref[...], preferred_element_type=jnp.float32)
    gi = gi + bih_ref[...]
    gh = jnp.dot(w0_ref[...], whh_ref[...], preferred_element_type=jnp.float32)
    gh = gh + bhh_ref[...]
    r = jax.nn.sigmoid(gi[:, :_F] + gh[:, :_F])
    z = jax.nn.sigmoid(gi[:, _F:2 * _F] + gh[:, _F:2 * _F])
    n = jnp.tanh(gi[:, 2 * _F:] + r * gh[:, 2 * _F:])
    w = (1.0 - z) * n + z * w0_ref[...]
    xw_ref[...] = jnp.dot(xp_ref[...], w, preferred_element_type=jnp.float32)


def _dense_call(s80, xp, dp3, wihT, whhT, bih2, bhh2, w0):
    nr = _NP // 128
    return pl.pallas_call(
        _dense_body,
        out_shape=[
            jax.ShapeDtypeStruct((_NP, _F), jnp.float32),
            jax.ShapeDtypeStruct((nr, 128), jnp.float32),
            jax.ShapeDtypeStruct((nr, 128), jnp.float32),
        ],
        scratch_shapes=[pltpu.VMEM((_F, _F), jnp.float32)],
    )(s80, xp, dp3, wihT, whhT, bih2, bhh2, w0)


def _final_body(acc_ref, xw_ref, sw_ref, wl_ref, bl_ref, o_ref):
    out = acc_ref[0] + acc_ref[1] + sw_ref[...] * xw_ref[...]
    h = jnp.maximum(out, 0.0)
    o_ref[...] = jnp.dot(h, wl_ref[...],
                         preferred_element_type=jnp.float32) + bl_ref[...]


def _final_call(acc, xw, sw, wlT, bl2):
    return pl.pallas_call(
        _final_body,
        out_shape=jax.ShapeDtypeStruct((_NP, wlT.shape[1]), jnp.float32),
    )(acc, xw, sw, wlT, bl2)


def kernel(x, edge_index, edge_weight, p, W_ih, W_hh, b_ih, b_hh, W0,
           W_lin, b_lin):
    n, f = x.shape
    e = edge_weight.shape[0]
    rpt = -(-e // (_NW * 128))
    rpt = -(-rpt // (2 * _BLK)) * (2 * _BLK)
    e_pad = _NW * 128 * rpt
    fill = jnp.arange(e_pad - e, dtype=jnp.int32) % n
    src = jnp.concatenate([edge_index[0], fill])
    dst = jnp.concatenate([edge_index[1], fill])
    w = jnp.concatenate([edge_weight,
                         jnp.zeros((e_pad - e,), edge_weight.dtype)])
    tr = _NW * rpt
    src2d = src.reshape(tr, 128)
    dst2d = dst.reshape(tr, 128)
    w2d = w.reshape(tr, 128)

    xp = jnp.pad(x, ((0, _NP - n), (0, 0)))

    score = _score_call(x, p.reshape(f, 1))
    s80 = jnp.pad(score, ((0, _NP - n), (0, 0))).reshape(_NP // 128, 128)

    dp = _deg_call(dst2d, w2d, rpt)

    xw, dis80, sw80 = _dense_call(
        s80, xp, dp.reshape(_NC, _NP // 128, 128),
        W_ih.T, W_hh.T, b_ih.reshape(1, -1), b_hh.reshape(1, -1), W0)

    acc = _msg_call(xw, dis80.reshape(_NP), src2d, dst2d, w2d, rpt)

    y = _final_call(acc, xw, sw80.reshape(_NP, 1), W_lin.T,
                    b_lin.reshape(1, -1))
    return y[:n]
